# baseline (speedup 1.0000x reference)
import functools

import jax
import jax.numpy as jnp
from jax import lax
from jax.experimental import pallas as pl
from jax.experimental.pallas import tpu as pltpu
from jax.experimental.pallas import tpu_sc as plsc


@functools.lru_cache(maxsize=None)
def _build_repack(D, V):
    info = plsc.get_sparse_core_info()
    NC, NS, L = info.num_cores, info.num_subcores, info.num_lanes
    NW = NC * NS
    nch = (V + 127) // 128
    P = nch * 64
    mesh = plsc.VectorSubcoreMesh(core_axis_name="c", subcore_axis_name="s")

    @functools.partial(
        pl.kernel,
        mesh=mesh,
        compiler_params=pltpu.CompilerParams(
            use_tc_tiling_on_sc=True, needs_layout_passes=False,
            disable_bounds_checks=True),
        out_type=jax.ShapeDtypeStruct((P, 128), jnp.float32),
        scratch_types=[
            pltpu.VMEM((2, D, 128), jnp.float32),
            pltpu.VMEM((2, D, 128), jnp.float32),
            pltpu.SemaphoreType.DMA,
            pltpu.SemaphoreType.DMA,
            pltpu.SemaphoreType.DMA,
            pltpu.SemaphoreType.DMA,
        ],
    )
    def body(tt_hbm, out_hbm, sblk_v, outc_v, g0, g1, w0, w1):
        wid = lax.axis_index("s") * NC + lax.axis_index("c")
        my_n = (nch - wid + NW - 1) // NW
        gsems = (g0, g1)
        wsems = (w0, w1)
        iot = lax.iota(jnp.int32, L)
        rotcs = [(iot + k) & (L - 1) for k in range(L)]
        colc = [rotcs[k] * 128 + 2 * iot for k in range(L)]

        def fire_read(t, buf):
            c = wid + t * NW
            pltpu.async_copy(tt_hbm.at[:, pl.ds(c * 128, 128)],
                             sblk_v.at[buf], gsems[buf])

        def drain_read(buf):
            pltpu.make_async_copy(tt_hbm.at[:, pl.ds(0, 128)],
                                  sblk_v.at[buf], gsems[buf]).wait()

        def fire_write(t, buf):
            c = wid + t * NW
            pltpu.async_copy(outc_v.at[buf], out_hbm.at[pl.ds(c * 64, 64), :],
                             wsems[buf])

        def drain_write(buf):
            pltpu.make_async_copy(outc_v.at[buf],
                                  out_hbm.at[pl.ds(0, 64), :],
                                  wsems[buf]).wait()

        def repack(buf):
            sref = sblk_v.at[buf]
            oref = outc_v.at[buf]

            def per_r(r, carry):
                prow = iot + r * 16
                for h2 in range(0, 8, 2):
                    pend = []
                    for h in (h2, h2 + 1):
                        rowb = (h % 4) * 16
                        colb = 2 * (r * 16) + (h // 4)
                        for k in range(L):
                            x = plsc.load_gather(
                                sref, [rotcs[k] + rowb, 2 * iot + colb])
                            pend.append((h, k, x))
                    for h, k, x in pend:
                        plsc.store_scatter(
                            oref, [prow, rotcs[k] + h * 16], x)
                return carry

            lax.fori_loop(0, D // L, per_r, 0)

        fire_read(0, 0)

        @pl.when(my_n > 1)
        def _():
            fire_read(1, 1)

        def step2(t2, carry):
            t0 = 2 * t2

            def one(t, buf):
                @pl.when(t < my_n)
                def _():
                    drain_read(buf)

                    @pl.when(t >= 2)
                    def _():
                        drain_write(buf)

                    repack(buf)
                    fire_write(t, buf)

                    @pl.when(t + 2 < my_n)
                    def _():
                        fire_read(t + 2, buf)

            one(t0, 0)
            one(t0 + 1, 1)
            return carry

        lax.fori_loop(0, (my_n + 1) // 2, step2, 0)

        @pl.when(my_n >= 1)
        def _():
            drain_write(0)

        @pl.when(my_n >= 2)
        def _():
            drain_write(1)

    return body


@functools.lru_cache(maxsize=None)
def _build_gather(B, S, D, V2):
    info = plsc.get_sparse_core_info()
    NC, NS, L = info.num_cores, info.num_subcores, info.num_lanes
    NW = NC * NS
    assert B % (NW * 128) == 0 and B // (NW * 128) == 1
    assert D % L == 0 and S % 2 == 0
    QF = D // L

    mesh = plsc.VectorSubcoreMesh(core_axis_name="c", subcore_axis_name="s")

    @functools.partial(
        pl.kernel,
        mesh=mesh,
        compiler_params=pltpu.CompilerParams(
            use_tc_tiling_on_sc=True, needs_layout_passes=False,
            disable_bounds_checks=True),
        out_type=jax.ShapeDtypeStruct((S, D, B), jnp.float32),
        scratch_types=[
            pltpu.VMEM((S, 128), jnp.int32),
            pltpu.VMEM((S, 128), jnp.int32),
            pltpu.VMEM((S * D,), jnp.float32),
            pltpu.VMEM((2, 128, 2 * D), jnp.float32),
            pltpu.VMEM((2, D, 128), jnp.float32),
            pltpu.SemaphoreType.DMA,
            pltpu.SemaphoreType.DMA,
            pltpu.SemaphoreType.DMA,
            pltpu.SemaphoreType.DMA,
        ],
    )
    def body(idx_hbm, tab_hbm, pe_hbm, out_hbm,
             idx_v, gidx_v, pe_v, rows_v, outb_v, g0, g1, w0, w1):
        wid = lax.axis_index("s") * NC + lax.axis_index("c")
        b0 = wid * 128
        pltpu.sync_copy(idx_hbm.at[:, pl.ds(b0, 128)], idx_v)
        pltpu.sync_copy(pe_hbm, pe_v)

        def mk_gidx(s, carry):
            for g in range(8):
                w = idx_v[s, pl.ds(g * L, L)]
                gidx_v[s, pl.ds(g * L, L)] = lax.shift_right_logical(w, 1)
            return carry

        lax.fori_loop(0, S, mk_gidx, 0)

        jvs = [lax.iota(jnp.int32, L) + g * L for g in range(8)]
        gsems = (g0, g1)
        wsems = (w0, w1)

        def fire_gather(srow, buf):
            pltpu.async_copy(tab_hbm.at[gidx_v.at[srow]], rows_v.at[buf],
                             gsems[buf])

        def drain_gather(buf):
            pltpu.make_async_copy(tab_hbm.at[gidx_v.at[0]], rows_v.at[buf],
                                  gsems[buf]).wait()

        def fire_write(s, buf):
            pltpu.async_copy(outb_v.at[buf], out_hbm.at[s, :, pl.ds(b0, 128)],
                             wsems[buf])

        def drain_write(buf):
            pltpu.make_async_copy(outb_v.at[buf],
                                  out_hbm.at[0, :, pl.ds(b0, 128)],
                                  wsems[buf]).wait()

        iot = lax.iota(jnp.int32, L)
        rotcs = [(iot + k) & (L - 1) for k in range(L)]

        def block(s, buf):
            colbs = []
            for g in range(8):
                w = idx_v[s, pl.ds(g * L, L)]
                colbs.append(lax.shift_left(w & 1, 6))
            rows_ref = rows_v.at[buf]
            outb_ref = outb_v.at[buf]

            def per_fc(fc, cbs):
                fb = fc * L
                peb = lax.broadcast(s * D + fb, (L,))
                for k4 in range(0, L, 4):
                    pend = []
                    for k in range(k4, k4 + 4):
                        rfk = fb + rotcs[k]
                        pef = plsc.load_gather(pe_v, [peb + rotcs[k]])
                        for g in range(8):
                            x = plsc.load_gather(
                                rows_ref, [jvs[g], cbs[g] + rfk])
                            pend.append((rfk, g, x, pef))
                    for rfk, g, x, pef in pend:
                        plsc.store_scatter(outb_ref, [rfk, jvs[g]], x + pef)
                return cbs

            lax.fori_loop(0, D // L, per_fc, tuple(colbs))

        fire_gather(0, 0)

        def step(k, carry):
            s0 = 2 * k
            s1 = 2 * k + 1
            fire_gather(s1, 1)
            drain_gather(0)

            @pl.when(k > 0)
            def _():
                drain_write(0)

            block(s0, 0)
            fire_write(s0, 0)

            s2 = jnp.minimum(s0 + 2, S - 1)
            fire_gather(s2, 0)
            drain_gather(1)

            @pl.when(k > 0)
            def _():
                drain_write(1)

            block(s1, 1)
            fire_write(s1, 1)
            return carry

        lax.fori_loop(0, S // 2, step, 0)
        drain_gather(0)
        drain_write(0)
        drain_write(1)

    return body


def kernel(inputs, table, pos_encoding):
    B, S = inputs.shape
    V, D = table.shape
    idx_t = inputs.T.astype(jnp.int32)
    table2 = _build_repack(D, V)(table.T)
    pe = pos_encoding[:S].astype(jnp.float32).reshape(-1)
    out3 = _build_gather(B, S, D, table2.shape[0])(idx_t, table2, pe)
    return out3.transpose(2, 0, 1)


# speedup vs baseline: 1.0494x; 1.0494x over previous
import functools

import jax
import jax.numpy as jnp
from jax import lax
from jax.experimental import pallas as pl
from jax.experimental.pallas import tpu as pltpu
from jax.experimental.pallas import tpu_sc as plsc


@functools.lru_cache(maxsize=None)
def _build_repack(D, V):
    info = plsc.get_sparse_core_info()
    NC, NS, L = info.num_cores, info.num_subcores, info.num_lanes
    NW = NC * NS
    nch = (V + 127) // 128
    P = nch * 64
    mesh = plsc.VectorSubcoreMesh(core_axis_name="c", subcore_axis_name="s")

    @functools.partial(
        pl.kernel,
        mesh=mesh,
        compiler_params=pltpu.CompilerParams(
            use_tc_tiling_on_sc=True, needs_layout_passes=False,
            disable_bounds_checks=True),
        out_type=jax.ShapeDtypeStruct((P, 128), jnp.float32),
        scratch_types=[
            pltpu.VMEM((2, D, 128), jnp.float32),
            pltpu.VMEM((2, D, 128), jnp.float32),
            pltpu.SemaphoreType.DMA,
            pltpu.SemaphoreType.DMA,
            pltpu.SemaphoreType.DMA,
            pltpu.SemaphoreType.DMA,
        ],
    )
    def body(tt_hbm, out_hbm, sblk_v, outc_v, g0, g1, w0, w1):
        wid = lax.axis_index("s") * NC + lax.axis_index("c")
        my_n = (nch - wid + NW - 1) // NW
        gsems = (g0, g1)
        wsems = (w0, w1)
        iot = lax.iota(jnp.int32, L)
        rotcs = [(iot + k) & (L - 1) for k in range(L)]
        colc = [rotcs[k] * 128 + 2 * iot for k in range(L)]

        def fire_read(t, buf):
            c = wid + t * NW
            pltpu.async_copy(tt_hbm.at[:, pl.ds(c * 128, 128)],
                             sblk_v.at[buf], gsems[buf])

        def drain_read(buf):
            pltpu.make_async_copy(tt_hbm.at[:, pl.ds(0, 128)],
                                  sblk_v.at[buf], gsems[buf]).wait()

        def fire_write(t, buf):
            c = wid + t * NW
            pltpu.async_copy(outc_v.at[buf], out_hbm.at[pl.ds(c * 64, 64), :],
                             wsems[buf])

        def drain_write(buf):
            pltpu.make_async_copy(outc_v.at[buf],
                                  out_hbm.at[pl.ds(0, 64), :],
                                  wsems[buf]).wait()

        def repack(buf):
            sref = sblk_v.at[buf]
            oref = outc_v.at[buf]

            def per_r(r, carry):
                prow = iot + r * 16
                for h in range(8):
                    rowb = (h % 4) * 16
                    colb = 2 * (r * 16) + (h // 4)
                    pend = []
                    for k in range(L):
                        x = plsc.load_gather(
                            sref, [rotcs[k] + rowb, 2 * iot + colb])
                        pend.append((k, x))
                    for k, x in pend:
                        plsc.store_scatter(
                            oref, [prow, rotcs[k] + h * 16], x)
                return carry

            lax.fori_loop(0, D // L, per_r, 0)

        fire_read(0, 0)

        @pl.when(my_n > 1)
        def _():
            fire_read(1, 1)

        def step2(t2, carry):
            t0 = 2 * t2

            def one(t, buf):
                @pl.when(t < my_n)
                def _():
                    drain_read(buf)

                    @pl.when(t >= 2)
                    def _():
                        drain_write(buf)

                    repack(buf)
                    fire_write(t, buf)

                    @pl.when(t + 2 < my_n)
                    def _():
                        fire_read(t + 2, buf)

            one(t0, 0)
            one(t0 + 1, 1)
            return carry

        lax.fori_loop(0, (my_n + 1) // 2, step2, 0)

        @pl.when(my_n >= 1)
        def _():
            drain_write(0)

        @pl.when(my_n >= 2)
        def _():
            drain_write(1)

    return body


@functools.lru_cache(maxsize=None)
def _build_gather(B, S, D, V2):
    info = plsc.get_sparse_core_info()
    NC, NS, L = info.num_cores, info.num_subcores, info.num_lanes
    NW = NC * NS
    assert B % (NW * 128) == 0 and B // (NW * 128) == 1
    assert D % L == 0 and S % 2 == 0
    QF = D // L

    mesh = plsc.VectorSubcoreMesh(core_axis_name="c", subcore_axis_name="s")

    @functools.partial(
        pl.kernel,
        mesh=mesh,
        compiler_params=pltpu.CompilerParams(
            use_tc_tiling_on_sc=True, needs_layout_passes=False,
            disable_bounds_checks=True),
        out_type=jax.ShapeDtypeStruct((S, D, B), jnp.float32),
        scratch_types=[
            pltpu.VMEM((S, 128), jnp.int32),
            pltpu.VMEM((S, 128), jnp.int32),
            pltpu.VMEM((S * D,), jnp.float32),
            pltpu.VMEM((2, 128, 2 * D), jnp.float32),
            pltpu.VMEM((2, D, 128), jnp.float32),
            pltpu.SemaphoreType.DMA,
            pltpu.SemaphoreType.DMA,
            pltpu.SemaphoreType.DMA,
            pltpu.SemaphoreType.DMA,
        ],
    )
    def body(idx_hbm, tab_hbm, pe_hbm, out_hbm,
             idx_v, gidx_v, pe_v, rows_v, outb_v, g0, g1, w0, w1):
        wid = lax.axis_index("s") * NC + lax.axis_index("c")
        b0 = wid * 128
        pltpu.sync_copy(idx_hbm.at[:, pl.ds(b0, 128)], idx_v)
        pltpu.sync_copy(pe_hbm, pe_v)

        def mk_gidx(s, carry):
            for g in range(8):
                w = idx_v[s, pl.ds(g * L, L)]
                gidx_v[s, pl.ds(g * L, L)] = lax.shift_right_logical(w, 1)
            return carry

        lax.fori_loop(0, S, mk_gidx, 0)

        jvs = [lax.iota(jnp.int32, L) + g * L for g in range(8)]
        gsems = (g0, g1)
        wsems = (w0, w1)

        def fire_gather(srow, buf):
            pltpu.async_copy(tab_hbm.at[gidx_v.at[srow]], rows_v.at[buf],
                             gsems[buf])

        def drain_gather(buf):
            pltpu.make_async_copy(tab_hbm.at[gidx_v.at[0]], rows_v.at[buf],
                                  gsems[buf]).wait()

        def fire_write(s, buf):
            pltpu.async_copy(outb_v.at[buf], out_hbm.at[s, :, pl.ds(b0, 128)],
                             wsems[buf])

        def drain_write(buf):
            pltpu.make_async_copy(outb_v.at[buf],
                                  out_hbm.at[0, :, pl.ds(b0, 128)],
                                  wsems[buf]).wait()

        iot = lax.iota(jnp.int32, L)
        rotcs = [(iot + k) & (L - 1) for k in range(L)]

        def block(s, buf):
            colbs = []
            for g in range(8):
                w = idx_v[s, pl.ds(g * L, L)]
                colbs.append(lax.shift_left(w & 1, 6))
            rows_ref = rows_v.at[buf]
            outb_ref = outb_v.at[buf]

            def per_fc(fc, cbs):
                fb = fc * L
                peb = lax.broadcast(s * D + fb, (L,))
                for k4 in range(0, L, 4):
                    pend = []
                    for k in range(k4, k4 + 4):
                        rfk = fb + rotcs[k]
                        pef = plsc.load_gather(pe_v, [peb + rotcs[k]])
                        for g in range(8):
                            x = plsc.load_gather(
                                rows_ref, [jvs[g], cbs[g] + rfk])
                            pend.append((rfk, g, x, pef))
                    for rfk, g, x, pef in pend:
                        plsc.store_scatter(outb_ref, [rfk, jvs[g]], x + pef)
                return cbs

            lax.fori_loop(0, D // L, per_fc, tuple(colbs))

        fire_gather(0, 0)

        def step(k, carry):
            s0 = 2 * k
            s1 = 2 * k + 1
            fire_gather(s1, 1)
            drain_gather(0)

            @pl.when(k > 0)
            def _():
                drain_write(0)

            block(s0, 0)
            fire_write(s0, 0)

            s2 = jnp.minimum(s0 + 2, S - 1)
            fire_gather(s2, 0)
            drain_gather(1)

            @pl.when(k > 0)
            def _():
                drain_write(1)

            block(s1, 1)
            fire_write(s1, 1)
            return carry

        lax.fori_loop(0, S // 2, step, 0)
        drain_gather(0)
        drain_write(0)
        drain_write(1)

    return body


def kernel(inputs, table, pos_encoding):
    B, S = inputs.shape
    V, D = table.shape
    idx_t = inputs.T.astype(jnp.int32)
    table2 = _build_repack(D, V)(table.T)
    pe = pos_encoding[:S].astype(jnp.float32).reshape(-1)
    out3 = _build_gather(B, S, D, table2.shape[0])(idx_t, table2, pe)
    return out3.transpose(2, 0, 1)


# speedup vs baseline: 1.0526x; 1.0030x over previous
import functools

import jax
import jax.numpy as jnp
from jax import lax
from jax.experimental import pallas as pl
from jax.experimental.pallas import tpu as pltpu
from jax.experimental.pallas import tpu_sc as plsc


@functools.lru_cache(maxsize=None)
def _build_repack(D, V):
    info = plsc.get_sparse_core_info()
    NC, NS, L = info.num_cores, info.num_subcores, info.num_lanes
    NW = NC * NS
    nch = (V + 127) // 128
    P = nch * 64
    mesh = plsc.VectorSubcoreMesh(core_axis_name="c", subcore_axis_name="s")

    @functools.partial(
        pl.kernel,
        mesh=mesh,
        compiler_params=pltpu.CompilerParams(
            use_tc_tiling_on_sc=True, needs_layout_passes=False,
            disable_bounds_checks=True),
        out_type=jax.ShapeDtypeStruct((P, 128), jnp.float32),
        scratch_types=[
            pltpu.VMEM((2, D, 128), jnp.float32),
            pltpu.VMEM((2, D, 128), jnp.float32),
            pltpu.SemaphoreType.DMA,
            pltpu.SemaphoreType.DMA,
            pltpu.SemaphoreType.DMA,
            pltpu.SemaphoreType.DMA,
        ],
    )
    def body(tt_hbm, out_hbm, sblk_v, outc_v, g0, g1, w0, w1):
        wid = lax.axis_index("s") * NC + lax.axis_index("c")
        my_n = (nch - wid + NW - 1) // NW
        gsems = (g0, g1)
        wsems = (w0, w1)
        iot = lax.iota(jnp.int32, L)
        rotcs = [(iot + k) & (L - 1) for k in range(L)]
        colc = [rotcs[k] * 128 + 2 * iot for k in range(L)]

        def fire_read(t, buf):
            c = wid + t * NW
            pltpu.async_copy(tt_hbm.at[:, pl.ds(c * 128, 128)],
                             sblk_v.at[buf], gsems[buf])

        def drain_read(buf):
            pltpu.make_async_copy(tt_hbm.at[:, pl.ds(0, 128)],
                                  sblk_v.at[buf], gsems[buf]).wait()

        def fire_write(t, buf):
            c = wid + t * NW
            pltpu.async_copy(outc_v.at[buf], out_hbm.at[pl.ds(c * 64, 64), :],
                             wsems[buf])

        def drain_write(buf):
            pltpu.make_async_copy(outc_v.at[buf],
                                  out_hbm.at[pl.ds(0, 64), :],
                                  wsems[buf]).wait()

        def repack(buf):
            sref = sblk_v.at[buf]
            oref = outc_v.at[buf]

            def per_r(r, carry):
                prow = iot + r * 16
                for h in range(8):
                    rowb = (h % 4) * 16
                    colb = 2 * (r * 16) + (h // 4)
                    pend = []
                    for k in range(L):
                        x = plsc.load_gather(
                            sref, [rotcs[k] + rowb, 2 * iot + colb])
                        pend.append((k, x))
                    for k, x in pend:
                        plsc.store_scatter(
                            oref, [prow, rotcs[k] + h * 16], x)
                return carry

            lax.fori_loop(0, D // L, per_r, 0)

        fire_read(0, 0)

        @pl.when(my_n > 1)
        def _():
            fire_read(1, 1)

        def step2(t2, carry):
            t0 = 2 * t2

            def one(t, buf):
                @pl.when(t < my_n)
                def _():
                    drain_read(buf)

                    @pl.when(t >= 2)
                    def _():
                        drain_write(buf)

                    repack(buf)
                    fire_write(t, buf)

                    @pl.when(t + 2 < my_n)
                    def _():
                        fire_read(t + 2, buf)

            one(t0, 0)
            one(t0 + 1, 1)
            return carry

        lax.fori_loop(0, (my_n + 1) // 2, step2, 0)

        @pl.when(my_n >= 1)
        def _():
            drain_write(0)

        @pl.when(my_n >= 2)
        def _():
            drain_write(1)

    return body


@functools.lru_cache(maxsize=None)
def _build_gather(B, S, D, V2):
    info = plsc.get_sparse_core_info()
    NC, NS, L = info.num_cores, info.num_subcores, info.num_lanes
    NW = NC * NS
    assert B % (NW * 128) == 0 and B // (NW * 128) == 1
    assert D % L == 0 and S % 2 == 0

    mesh = plsc.VectorSubcoreMesh(core_axis_name="c", subcore_axis_name="s")

    @functools.partial(
        pl.kernel,
        mesh=mesh,
        compiler_params=pltpu.CompilerParams(
            use_tc_tiling_on_sc=True, needs_layout_passes=False,
            disable_bounds_checks=True),
        out_type=jax.ShapeDtypeStruct((S, D, B), jnp.float32),
        scratch_types=[
            pltpu.VMEM((S, 128), jnp.int32),
            pltpu.VMEM((S, 128), jnp.int32),
            pltpu.VMEM((S * D,), jnp.float32),
            pltpu.VMEM((2, 128, 2 * D), jnp.float32),
            pltpu.VMEM((2, D, 128), jnp.float32),
            pltpu.SemaphoreType.DMA,
            pltpu.SemaphoreType.DMA,
            pltpu.SemaphoreType.DMA,
            pltpu.SemaphoreType.DMA,
        ],
    )
    def body(idx_hbm, tab_hbm, pe_hbm, out_hbm,
             idx_v, gidx_v, pe_v, rows_v, outb_v, g0, g1, w0, w1):
        wid = lax.axis_index("s") * NC + lax.axis_index("c")
        b0 = wid * 128
        pltpu.sync_copy(idx_hbm.at[:, pl.ds(b0, 128)], idx_v)
        pltpu.sync_copy(pe_hbm, pe_v)

        def mk_gidx(s, carry):
            for g in range(8):
                w = idx_v[s, pl.ds(g * L, L)]
                gidx_v[s, pl.ds(g * L, L)] = lax.shift_right_logical(w, 1)
            return carry

        lax.fori_loop(0, S, mk_gidx, 0)

        jvs = [lax.iota(jnp.int32, L) + g * L for g in range(8)]
        gsems = (g0, g1)
        wsems = (w0, w1)

        def fire_gather(srow, buf):
            pltpu.async_copy(tab_hbm.at[gidx_v.at[srow]], rows_v.at[buf],
                             gsems[buf])

        def drain_gather(buf):
            pltpu.make_async_copy(tab_hbm.at[gidx_v.at[0]], rows_v.at[buf],
                                  gsems[buf]).wait()

        def fire_write(s, buf):
            pltpu.async_copy(outb_v.at[buf], out_hbm.at[s, :, pl.ds(b0, 128)],
                             wsems[buf])

        def drain_write(buf):
            pltpu.make_async_copy(outb_v.at[buf],
                                  out_hbm.at[0, :, pl.ds(b0, 128)],
                                  wsems[buf]).wait()

        iot = lax.iota(jnp.int32, L)
        rotcs = [(iot + k) & (L - 1) for k in range(L)]

        def block(s, buf):
            colbs = []
            for g in range(8):
                w = idx_v[s, pl.ds(g * L, L)]
                colbs.append(lax.shift_left(w & 1, 6))
            rows_ref = rows_v.at[buf]
            outb_ref = outb_v.at[buf]

            def per_fc(fc, cbs):
                fb = fc * L
                peb = lax.broadcast(s * D + fb, (L,))
                for k4 in range(0, L, 4):
                    pend = []
                    for k in range(k4, k4 + 4):
                        rfk = fb + rotcs[k]
                        pef = plsc.load_gather(pe_v, [peb + rotcs[k]])
                        for g in range(8):
                            x = plsc.load_gather(
                                rows_ref, [jvs[g], cbs[g] + rfk])
                            pend.append((rfk, g, x, pef))
                    for rfk, g, x, pef in pend:
                        plsc.store_scatter(outb_ref, [rfk, jvs[g]], x + pef)
                return cbs

            lax.fori_loop(0, D // L, per_fc, tuple(colbs))

        fire_gather(0, 0)

        def step(k, carry):
            s0 = 2 * k
            s1 = 2 * k + 1
            fire_gather(s1, 1)
            drain_gather(0)

            @pl.when(k > 0)
            def _():
                drain_write(0)

            block(s0, 0)
            fire_write(s0, 0)

            s2 = jnp.minimum(s0 + 2, S - 1)
            fire_gather(s2, 0)
            drain_gather(1)

            @pl.when(k > 0)
            def _():
                drain_write(1)

            block(s1, 1)
            fire_write(s1, 1)
            return carry

        lax.fori_loop(0, S // 2, step, 0)
        drain_gather(0)
        drain_write(0)
        drain_write(1)

    return body


def kernel(inputs, table, pos_encoding):
    B, S = inputs.shape
    V, D = table.shape
    idx_t = inputs.T.astype(jnp.int32)
    table2 = _build_repack(D, V)(table.T)
    pe = pos_encoding[:S].astype(jnp.float32).reshape(-1)
    out3 = _build_gather(B, S, D, table2.shape[0])(idx_t, table2, pe)
    return out3.transpose(2, 0, 1)
